# baseline (speedup 1.0000x reference)
import functools

import jax
import jax.numpy as jnp
from jax import lax
from jax.experimental import pallas as pl
from jax.experimental.pallas import tpu as pltpu
from jax.experimental.pallas import tpu_sc as plsc

_NC = 2
_NS = 16
_NW = _NC * _NS

_CIMG = 2
_NSLOT = 3
_NDRV = 4
_RA = 2


def _sc_body(n_chunks, in_hbm, out_hbm, bufs, rsems, wsems):
    c = lax.axis_index("c")
    s = lax.axis_index("s")

    for d in range(_NDRV):
        @pl.when(s == d)
        def _(d=d):
            chunks = list(range(d, n_chunks, _NDRV))

            def img0(g):
                return (c * n_chunks + g) * _CIMG

            def read(g, slot):
                pltpu.async_copy(
                    in_hbm.at[pl.ds(img0(g), _CIMG)], bufs[slot], rsems[slot])

            def wait_read(g, slot):
                pltpu.make_async_copy(
                    in_hbm.at[pl.ds(img0(g), _CIMG)], bufs[slot],
                    rsems[slot]).wait()

            def write(g, slot):
                pltpu.async_copy(
                    bufs[slot], out_hbm.at[pl.ds(img0(g), _CIMG)], wsems[slot])

            def wait_write(g, slot):
                pltpu.make_async_copy(
                    bufs[slot], out_hbm.at[pl.ds(img0(g), _CIMG)],
                    wsems[slot]).wait()

            base_slot = d * _NSLOT
            n = len(chunks)
            ra = min(_RA, n)
            for k in range(ra):
                read(chunks[k], base_slot + k % _NSLOT)
            for k in range(n):
                slot = base_slot + k % _NSLOT
                wait_read(chunks[k], slot)
                write(chunks[k], slot)
                nk = k + ra
                if nk < n:
                    conflict = nk - _NSLOT
                    if conflict >= 0:
                        wait_write(chunks[conflict],
                                   base_slot + conflict % _NSLOT)
                    read(chunks[nk], base_slot + nk % _NSLOT)
            for k in range(max(0, n - _NSLOT), n):
                wait_write(chunks[k], base_slot + k % _NSLOT)


def kernel(phi):
    B, H, W = phi.shape
    assert B % (_NC * _CIMG) == 0
    n_chunks = B // (_NC * _CIMG)

    mesh = plsc.VectorSubcoreMesh(core_axis_name="c", subcore_axis_name="s")
    run = pl.kernel(
        functools.partial(_sc_body, n_chunks),
        mesh=mesh,
        out_type=jax.ShapeDtypeStruct(phi.shape, phi.dtype),
        scratch_types=[
            [pltpu.VMEM_SHARED((_CIMG, H, W), jnp.float32)] * (_NSLOT * _NDRV),
            [pltpu.SemaphoreType.DMA] * (_NSLOT * _NDRV),
            [pltpu.SemaphoreType.DMA] * (_NSLOT * _NDRV),
        ],
    )
    return run(phi)
